# baseline (speedup 1.0000x reference)
import functools

import jax
import jax.numpy as jnp
from jax import lax
from jax.experimental import pallas as pl
from jax.experimental.pallas import tpu as pltpu
from jax.experimental.pallas import tpu_sc as plsc

NQ = 16
ND = 32
NCAND = 1000000
K = 100

BLK = 32768
NBLK = 31
NPAD = NBLK * BLK
CHUNK = 128
NCHUNK = NPAD // CHUNK
L2N = NCHUNK // 16
L3N = 32
P2_L2N = (K * CHUNK) // 16
P2_L2PAD = 1024
P2_L3N = 64

NEG = float("-inf")


def _score_body(q_ref, c_ref, scores_ref, cmax_ref):
    i = pl.program_id(0)
    q = q_ref[...]
    c = c_ref[...]
    s = lax.dot_general(q, c, (((1,), (0,)), ((), ())),
                        preferred_element_type=jnp.float32)

    @pl.when(i < NBLK - 1)
    def _full():
        s3 = s.reshape(NQ, BLK // CHUNK, CHUNK)
        scores_ref[...] = s3
        cmax_ref[...] = jnp.max(s3, axis=2)

    @pl.when(i == NBLK - 1)
    def _tail():
        gidx = i * BLK + lax.broadcasted_iota(jnp.int32, (NQ, BLK), 1)
        sm = jnp.where(gidx < NCAND, s, NEG)
        s3 = sm.reshape(NQ, BLK // CHUNK, CHUNK)
        scores_ref[...] = s3
        cmax_ref[...] = jnp.max(s3, axis=2)


def _stage_a(queries, candidates):
    return pl.pallas_call(
        _score_body,
        grid=(NBLK,),
        in_specs=[
            pl.BlockSpec((NQ, ND), lambda i: (0, 0)),
            pl.BlockSpec((ND, BLK), lambda i: (0, i)),
        ],
        out_specs=[
            pl.BlockSpec((NQ, BLK // CHUNK, CHUNK), lambda i: (0, i, 0)),
            pl.BlockSpec((NQ, BLK // CHUNK), lambda i: (0, i)),
        ],
        out_shape=[
            jax.ShapeDtypeStruct((NQ, NCHUNK, CHUNK), jnp.float32),
            jax.ShapeDtypeStruct((NQ, NCHUNK), jnp.float32),
        ],
        compiler_params=pltpu.CompilerParams(
            dimension_semantics=("arbitrary",)),
    )(queries, candidates)


def _ffs(mask):
    return jnp.max(plsc.all_reduce_ffs(mask))


def _hmax(v):
    return lax.reduce_max(v, (0,))


def _load1(ref, i):
    return jnp.max(plsc.load_gather(ref, [jnp.full((16,), i, jnp.int32)]))


def _store1(ref, i, val, lane):
    idx = jnp.full((16,), i, jnp.int32)
    v = jnp.full((16,), val)
    plsc.store_scatter(ref, [idx], v, mask=lane == 0)


def _sel_body(scores_hbm, cmax_hbm, ids_hbm, vals_out, idx_out,
              cm_buf, l2_buf, l3_buf, chunk_ids, score_buf,
              p2l2, p2l3, val_buf, idx_buf, gath_ids, sem):
    cid = lax.axis_index("c")
    sid = lax.axis_index("s")
    q = sid
    lane = lax.iota(jnp.int32, 16)
    neg16 = jnp.full((16,), NEG, jnp.float32)

    @pl.when(cid == 0)
    def _():
        pltpu.sync_copy(cmax_hbm.at[q], cm_buf)
        l2_buf[pl.ds(L2N, 16)] = neg16
        for g in range(8):
            chunk_ids[pl.ds(g * 16, 16)] = jnp.zeros((16,), jnp.int32)
            idx_buf[pl.ds(g * 16, 16)] = jnp.zeros((16,), jnp.int32)
            val_buf[pl.ds(g * 16, 16)] = jnp.zeros((16,), jnp.float32)

        def build_l2(g, _):
            acc = neg16
            for j in range(16):
                v = plsc.load_gather(cm_buf, [lane * 16 + g * 256 + j])
                acc = jnp.maximum(acc, v)
            l2_buf[pl.ds(g * 16, 16)] = acc
            return 0

        lax.fori_loop(0, L2N // 16, build_l2, 0)

        def build_l3(g, _):
            acc = neg16
            for j in range(16):
                v = plsc.load_gather(l2_buf, [lane * 16 + g * 256 + j])
                acc = jnp.maximum(acc, v)
            l3_buf[pl.ds(g * 16, 16)] = acc
            return 0

        lax.fori_loop(0, 2, build_l3, 0)

        def extract_chunk(t, _):
            v0 = l3_buf[pl.ds(0, 16)]
            v1 = l3_buf[pl.ds(16, 16)]
            m0 = _hmax(v0)
            m1 = _hmax(v1)
            use_hi = m1 > m0
            m = jnp.maximum(m0, m1)
            grp = jnp.where(use_hi, v1, v0)
            j = jnp.where(use_hi, 16, 0) + _ffs(grp == m)
            u = l2_buf[pl.ds(j * 16, 16)]
            i_off = _ffs(u == m)
            i = j * 16 + i_off
            w = cm_buf[pl.ds(i * 16, 16)]
            c_off = _ffs(w == m)
            _store1(chunk_ids, t, i * 16 + c_off, lane)
            w2 = jnp.where(lane == c_off, NEG, w)
            cm_buf[pl.ds(i * 16, 16)] = w2
            nv = jnp.full((16,), _hmax(w2), jnp.float32)
            u2 = jnp.where(lane == i_off, nv, u)
            l2_buf[pl.ds(j * 16, 16)] = u2
            _store1(l3_buf, j, _hmax(u2), lane)
            return 0

        lax.fori_loop(0, K, extract_chunk, 0)

        pltpu.async_copy(scores_hbm.at[q].at[chunk_ids], score_buf, sem).wait()

        for g in range(P2_L2N, P2_L2PAD, 16):
            p2l2[pl.ds(g, 16)] = neg16

        def build_p2l2(g, _):
            acc = neg16
            for j in range(16):
                f = lane * 16 + g * 256 + j
                v = plsc.load_gather(score_buf, [f >> 7, f & 127])
                acc = jnp.maximum(acc, v)
            p2l2[pl.ds(g * 16, 16)] = acc
            return 0

        lax.fori_loop(0, P2_L2N // 16, build_p2l2, 0)

        def build_p2l3(g, _):
            acc = neg16
            for j in range(16):
                v = plsc.load_gather(p2l2, [lane * 16 + g * 256 + j])
                acc = jnp.maximum(acc, v)
            p2l3[pl.ds(g * 16, 16)] = acc
            return 0

        lax.fori_loop(0, 4, build_p2l3, 0)

        def extract_elem(t, _):
            v0 = p2l3[pl.ds(0, 16)]
            v1 = p2l3[pl.ds(16, 16)]
            v2 = p2l3[pl.ds(32, 16)]
            v3 = p2l3[pl.ds(48, 16)]
            m0, m1, m2, m3 = _hmax(v0), _hmax(v1), _hmax(v2), _hmax(v3)
            m = jnp.maximum(jnp.maximum(m0, m1), jnp.maximum(m2, m3))
            g = jnp.where(m0 == m, 0,
                          jnp.where(m1 == m, 1, jnp.where(m2 == m, 2, 3)))
            grp = p2l3[pl.ds(g * 16, 16)]
            j = g * 16 + _ffs(grp == m)
            u = p2l2[pl.ds(j * 16, 16)]
            i_off = _ffs(u == m)
            e = j * 16 + i_off
            row = e >> 3
            col = (e & 7) * 16
            w = score_buf[row, pl.ds(col, 16)]
            c_off = _ffs(w == m)
            f = e * 16 + c_off
            _store1(val_buf, t, m, lane)
            _store1(idx_buf, t, _load1(chunk_ids, f >> 7) * CHUNK + (f & 127), lane)
            w2 = jnp.where(lane == c_off, NEG, w)
            score_buf[row, pl.ds(col, 16)] = w2
            nv = jnp.full((16,), _hmax(w2), jnp.float32)
            u2 = jnp.where(lane == i_off, nv, u)
            p2l2[pl.ds(j * 16, 16)] = u2
            _store1(p2l3, j, _hmax(u2), lane)
            return 0

        lax.fori_loop(0, K, extract_elem, 0)

        pltpu.async_copy(ids_hbm.at[idx_buf], gath_ids, sem).wait()
        pltpu.sync_copy(val_buf, vals_out.at[q])
        pltpu.sync_copy(gath_ids, idx_out.at[q])


def _stage_b(scores3, cmax, identifiers):
    mesh = plsc.VectorSubcoreMesh(core_axis_name="c", subcore_axis_name="s")
    kfn = pl.kernel(
        _sel_body,
        out_type=[
            jax.ShapeDtypeStruct((NQ, 128), jnp.float32),
            jax.ShapeDtypeStruct((NQ, 128), jnp.int32),
        ],
        mesh=mesh,
        scratch_types=[
            pltpu.VMEM((NCHUNK,), jnp.float32),
            pltpu.VMEM((L2N + 16,), jnp.float32),
            pltpu.VMEM((L3N,), jnp.float32),
            pltpu.VMEM((128,), jnp.int32),
            pltpu.VMEM((128, CHUNK), jnp.float32),
            pltpu.VMEM((P2_L2PAD,), jnp.float32),
            pltpu.VMEM((P2_L3N,), jnp.float32),
            pltpu.VMEM((128,), jnp.float32),
            pltpu.VMEM((128,), jnp.int32),
            pltpu.VMEM((128,), jnp.int32),
            pltpu.SemaphoreType.DMA,
        ],
        compiler_params=pltpu.CompilerParams(needs_layout_passes=False),
    )
    return kfn(scores3, cmax, identifiers)


def kernel(queries, candidates, identifiers, k):
    scores3, cmax = _stage_a(queries, candidates.T)
    vals, idx = _stage_b(scores3, cmax, identifiers)
    return (vals[:, :K], idx[:, :K])


# speedup vs baseline: 1.0726x; 1.0726x over previous
import functools

import jax
import jax.numpy as jnp
from jax import lax
from jax.experimental import pallas as pl
from jax.experimental.pallas import tpu as pltpu
from jax.experimental.pallas import tpu_sc as plsc

NQ = 16
ND = 32
NCAND = 1000000
K = 100

BLK = 32768
NBLK = 31
NPAD = NBLK * BLK
CHUNK = 128
NCHUNK = NPAD // CHUNK
L2N = NCHUNK // 16
L3N = 32
P2_L2N = (K * CHUNK) // 16
P2_L2PAD = 1024
P2_L3N = 64

NEG = float("-inf")


def _score_body(q_ref, c_ref, scores_ref, cmax_ref):
    i = pl.program_id(0)
    q = q_ref[...]
    c = c_ref[...]
    s = lax.dot_general(q, c, (((1,), (0,)), ((), ())),
                        preferred_element_type=jnp.float32)
    gidx = i * BLK + lax.broadcasted_iota(jnp.int32, (NQ, BLK), 1)
    s = jnp.where(gidx < NCAND, s, NEG)
    s3 = s.reshape(NQ, BLK // CHUNK, CHUNK)
    scores_ref[...] = s3
    cmax_ref[...] = jnp.max(s3, axis=2)


def _stage_a(queries, candidates):
    return pl.pallas_call(
        _score_body,
        grid=(NBLK,),
        in_specs=[
            pl.BlockSpec((NQ, ND), lambda i: (0, 0)),
            pl.BlockSpec((ND, BLK), lambda i: (0, i)),
        ],
        out_specs=[
            pl.BlockSpec((NQ, BLK // CHUNK, CHUNK), lambda i: (0, i, 0)),
            pl.BlockSpec((NQ, BLK // CHUNK), lambda i: (0, i)),
        ],
        out_shape=[
            jax.ShapeDtypeStruct((NQ, NCHUNK, CHUNK), jnp.float32),
            jax.ShapeDtypeStruct((NQ, NCHUNK), jnp.float32),
        ],
        compiler_params=pltpu.CompilerParams(
            dimension_semantics=("arbitrary",)),
    )(queries, candidates)


def _ffs(mask):
    return plsc.all_reduce_ffs(mask)[0]


def _pc(mask):
    return plsc.all_reduce_population_count(mask)[0]


def _hmax(v):
    return lax.reduce_max(v, (0,))


def _load1(ref, i):
    return plsc.load_gather(ref, [jnp.full((16,), i, jnp.int32)])[0]


def _store1(ref, i, val, lane):
    idx = jnp.full((16,), i, jnp.int32)
    v = jnp.full((16,), val)
    plsc.store_scatter(ref, [idx], v, mask=lane == 0)


def _sel_body(scores_hbm, cmax_hbm, ids_hbm, vals_out, idx_out,
              cm_buf, l2_buf, chunk_ids, score_buf,
              p2l2, val_buf, idx_buf, gath_ids, sem):
    cid = lax.axis_index("c")
    sid = lax.axis_index("s")
    q = sid
    lane = lax.iota(jnp.int32, 16)
    neg16 = jnp.full((16,), NEG, jnp.float32)

    @pl.when(cid == 0)
    def _():
        pltpu.sync_copy(cmax_hbm.at[q], cm_buf)
        l2_buf[pl.ds(L2N, 16)] = neg16
        for g in range(8):
            chunk_ids[pl.ds(g * 16, 16)] = jnp.zeros((16,), jnp.int32)
            idx_buf[pl.ds(g * 16, 16)] = jnp.zeros((16,), jnp.int32)
            val_buf[pl.ds(g * 16, 16)] = jnp.zeros((16,), jnp.float32)

        def build_l2(g, _):
            acc = neg16
            for j in range(16):
                v = plsc.load_gather(cm_buf, [lane * 16 + g * 256 + j])
                acc = jnp.maximum(acc, v)
            l2_buf[pl.ds(g * 16, 16)] = acc
            return 0

        lax.fori_loop(0, L2N // 16, build_l2, 0)

        def _l3_group(buf, g):
            acc = neg16
            for j in range(16):
                acc = jnp.maximum(
                    acc, plsc.load_gather(buf, [lane * 16 + g * 256 + j]))
            return acc

        v0 = _l3_group(l2_buf, 0)
        v1 = _l3_group(l2_buf, 1)

        def extract_chunk(t, carry):
            v0, v1 = carry
            m = _hmax(jnp.maximum(v0, v1))
            use_hi = _pc(v0 == m) == 0
            grp = jnp.where(use_hi, v1, v0)
            base = jnp.where(use_hi, 16, 0)
            j = base + _ffs(grp == m)
            u = l2_buf[pl.ds(j * 16, 16)]
            i_off = _ffs(u == m)
            i = j * 16 + i_off
            w = cm_buf[pl.ds(i * 16, 16)]
            c_off = _ffs(w == m)
            _store1(chunk_ids, t, i * 16 + c_off, lane)
            w2 = jnp.where(lane == c_off, NEG, w)
            cm_buf[pl.ds(i * 16, 16)] = w2
            u_masked = jnp.where(lane == i_off, NEG, u)
            nvw = _hmax(w2)
            nvu = jnp.maximum(_hmax(u_masked), nvw)
            l2_buf[pl.ds(j * 16, 16)] = jnp.where(lane == i_off, nvw, u)
            upd = lane == (j - base)
            v0n = jnp.where(jnp.logical_and(jnp.logical_not(use_hi), upd),
                            nvu, v0)
            v1n = jnp.where(jnp.logical_and(use_hi, upd), nvu, v1)
            return (v0n, v1n)

        lax.fori_loop(0, K, extract_chunk, (v0, v1))

        pltpu.async_copy(scores_hbm.at[q].at[chunk_ids], score_buf, sem).wait()

        for g in range(P2_L2N, P2_L2PAD, 16):
            p2l2[pl.ds(g, 16)] = neg16

        def build_p2l2(g, _):
            acc = neg16
            for j in range(16):
                f = lane * 16 + g * 256 + j
                v = plsc.load_gather(score_buf, [f >> 7, f & 127])
                acc = jnp.maximum(acc, v)
            p2l2[pl.ds(g * 16, 16)] = acc
            return 0

        lax.fori_loop(0, P2_L2N // 16, build_p2l2, 0)

        t0 = _l3_group(p2l2, 0)
        t1 = _l3_group(p2l2, 1)
        t2 = _l3_group(p2l2, 2)
        t3 = _l3_group(p2l2, 3)

        def extract_elem(t, carry):
            t0, t1, t2, t3 = carry
            m = _hmax(jnp.maximum(jnp.maximum(t0, t1), jnp.maximum(t2, t3)))
            in0 = _pc(t0 == m) > 0
            in1 = _pc(t1 == m) > 0
            in2 = _pc(t2 == m) > 0
            g = jnp.where(in0, 0, jnp.where(in1, 1, jnp.where(in2, 2, 3)))
            grp = jnp.where(in0, t0, jnp.where(in1, t1, jnp.where(in2, t2, t3)))
            j = g * 16 + _ffs(grp == m)
            u = p2l2[pl.ds(j * 16, 16)]
            i_off = _ffs(u == m)
            e = j * 16 + i_off
            row = e >> 3
            col = (e & 7) * 16
            w = score_buf[row, pl.ds(col, 16)]
            c_off = _ffs(w == m)
            f = e * 16 + c_off
            _store1(val_buf, t, m, lane)
            _store1(idx_buf, t,
                    _load1(chunk_ids, f >> 7) * CHUNK + (f & 127), lane)
            w2 = jnp.where(lane == c_off, NEG, w)
            score_buf[row, pl.ds(col, 16)] = w2
            u_masked = jnp.where(lane == i_off, NEG, u)
            nvw = _hmax(w2)
            nvu = jnp.maximum(_hmax(u_masked), nvw)
            p2l2[pl.ds(j * 16, 16)] = jnp.where(lane == i_off, nvw, u)
            upd = lane == (j - g * 16)
            t0n = jnp.where(jnp.logical_and(g == 0, upd), nvu, t0)
            t1n = jnp.where(jnp.logical_and(g == 1, upd), nvu, t1)
            t2n = jnp.where(jnp.logical_and(g == 2, upd), nvu, t2)
            t3n = jnp.where(jnp.logical_and(g == 3, upd), nvu, t3)
            return (t0n, t1n, t2n, t3n)

        lax.fori_loop(0, K, extract_elem, (t0, t1, t2, t3))

        pltpu.async_copy(ids_hbm.at[idx_buf], gath_ids, sem).wait()
        pltpu.sync_copy(val_buf, vals_out.at[q])
        pltpu.sync_copy(gath_ids, idx_out.at[q])


def _stage_b(scores3, cmax, identifiers):
    mesh = plsc.VectorSubcoreMesh(core_axis_name="c", subcore_axis_name="s")
    kfn = pl.kernel(
        _sel_body,
        out_type=[
            jax.ShapeDtypeStruct((NQ, 128), jnp.float32),
            jax.ShapeDtypeStruct((NQ, 128), jnp.int32),
        ],
        mesh=mesh,
        scratch_types=[
            pltpu.VMEM((NCHUNK,), jnp.float32),
            pltpu.VMEM((L2N + 16,), jnp.float32),
            pltpu.VMEM((128,), jnp.int32),
            pltpu.VMEM((128, CHUNK), jnp.float32),
            pltpu.VMEM((P2_L2PAD,), jnp.float32),
            pltpu.VMEM((128,), jnp.float32),
            pltpu.VMEM((128,), jnp.int32),
            pltpu.VMEM((128,), jnp.int32),
            pltpu.SemaphoreType.DMA,
        ],
        compiler_params=pltpu.CompilerParams(needs_layout_passes=False),
    )
    return kfn(scores3, cmax, identifiers)


def kernel(queries, candidates, identifiers, k):
    scores3, cmax = _stage_a(queries, candidates.T)
    vals, idx = _stage_b(scores3, cmax, identifiers)
    return (vals[:, :K], idx[:, :K])


# speedup vs baseline: 1.0879x; 1.0142x over previous
import functools

import jax
import jax.numpy as jnp
from jax import lax
from jax.experimental import pallas as pl
from jax.experimental.pallas import tpu as pltpu
from jax.experimental.pallas import tpu_sc as plsc

NQ = 16
ND = 32
NCAND = 1000000
K = 100

BLK = 65536
NBLK = 16
NPAD = NBLK * BLK
CHUNK = 128
NCHUNK = NPAD // CHUNK
L2N = NCHUNK // 16
L3N = 32
P2_L2N = (K * CHUNK) // 16
P2_L2PAD = 1024
P2_L3N = 64

NEG = float("-inf")


def _score_body(q_ref, c_ref, scores_ref, cmax_ref):
    i = pl.program_id(0)
    q = q_ref[...]
    c = c_ref[...]
    s = lax.dot_general(q, c, (((1,), (0,)), ((), ())),
                        preferred_element_type=jnp.float32)
    gidx = i * BLK + lax.broadcasted_iota(jnp.int32, (NQ, BLK), 1)
    s = jnp.where(gidx < NCAND, s, NEG)
    s3 = s.reshape(NQ, BLK // CHUNK, CHUNK)
    scores_ref[...] = s3
    cmax_ref[...] = jnp.max(s3, axis=2)


def _stage_a(queries, candidates):
    return pl.pallas_call(
        _score_body,
        grid=(NBLK,),
        in_specs=[
            pl.BlockSpec((NQ, ND), lambda i: (0, 0)),
            pl.BlockSpec((ND, BLK), lambda i: (0, i)),
        ],
        out_specs=[
            pl.BlockSpec((NQ, BLK // CHUNK, CHUNK), lambda i: (0, i, 0)),
            pl.BlockSpec((NQ, BLK // CHUNK), lambda i: (0, i)),
        ],
        out_shape=[
            jax.ShapeDtypeStruct((NQ, NCHUNK, CHUNK), jnp.float32),
            jax.ShapeDtypeStruct((NQ, NCHUNK), jnp.float32),
        ],
        compiler_params=pltpu.CompilerParams(
            dimension_semantics=("arbitrary",)),
    )(queries, candidates)


def _ffs(mask):
    return plsc.all_reduce_ffs(mask)[0]


def _pc(mask):
    return plsc.all_reduce_population_count(mask)[0]


def _hmax(v):
    return lax.reduce_max(v, (0,))


def _load1(ref, i):
    return plsc.load_gather(ref, [jnp.full((16,), i, jnp.int32)])[0]


def _store1(ref, i, val, lane):
    idx = jnp.full((16,), i, jnp.int32)
    v = jnp.full((16,), val)
    plsc.store_scatter(ref, [idx], v, mask=lane == 0)


def _sel_body(scores_hbm, cmax_hbm, ids_hbm, vals_out, idx_out,
              cm_buf, l2_buf, chunk_ids, score_buf,
              p2l2, val_buf, idx_buf, gath_ids, sem):
    cid = lax.axis_index("c")
    sid = lax.axis_index("s")
    q = sid
    lane = lax.iota(jnp.int32, 16)
    neg16 = jnp.full((16,), NEG, jnp.float32)

    @pl.when(cid == 0)
    def _():
        pltpu.sync_copy(cmax_hbm.at[q], cm_buf)
        l2_buf[pl.ds(L2N, 16)] = neg16
        for g in range(8):
            chunk_ids[pl.ds(g * 16, 16)] = jnp.zeros((16,), jnp.int32)
            idx_buf[pl.ds(g * 16, 16)] = jnp.zeros((16,), jnp.int32)
            val_buf[pl.ds(g * 16, 16)] = jnp.zeros((16,), jnp.float32)

        def build_l2(g, _):
            acc = neg16
            for j in range(16):
                v = plsc.load_gather(cm_buf, [lane * 16 + g * 256 + j])
                acc = jnp.maximum(acc, v)
            l2_buf[pl.ds(g * 16, 16)] = acc
            return 0

        lax.fori_loop(0, L2N // 16, build_l2, 0)

        def _l3_group(buf, g):
            acc = neg16
            for j in range(16):
                acc = jnp.maximum(
                    acc, plsc.load_gather(buf, [lane * 16 + g * 256 + j]))
            return acc

        v0 = _l3_group(l2_buf, 0)
        v1 = _l3_group(l2_buf, 1)

        def extract_chunk(t, carry):
            v0, v1 = carry
            m = _hmax(jnp.maximum(v0, v1))
            use_hi = _pc(v0 == m) == 0
            grp = jnp.where(use_hi, v1, v0)
            base = jnp.where(use_hi, 16, 0)
            j = base + _ffs(grp == m)
            u = l2_buf[pl.ds(j * 16, 16)]
            i_off = _ffs(u == m)
            i = j * 16 + i_off
            w = cm_buf[pl.ds(i * 16, 16)]
            c_off = _ffs(w == m)
            _store1(chunk_ids, t, i * 16 + c_off, lane)
            w2 = jnp.where(lane == c_off, NEG, w)
            cm_buf[pl.ds(i * 16, 16)] = w2
            u_masked = jnp.where(lane == i_off, NEG, u)
            nvw = _hmax(w2)
            nvu = jnp.maximum(_hmax(u_masked), nvw)
            l2_buf[pl.ds(j * 16, 16)] = jnp.where(lane == i_off, nvw, u)
            upd = lane == (j - base)
            v0n = jnp.where(jnp.logical_and(jnp.logical_not(use_hi), upd),
                            nvu, v0)
            v1n = jnp.where(jnp.logical_and(use_hi, upd), nvu, v1)
            return (v0n, v1n)

        lax.fori_loop(0, K, extract_chunk, (v0, v1))

        pltpu.async_copy(scores_hbm.at[q].at[chunk_ids], score_buf, sem).wait()

        for g in range(P2_L2N, P2_L2PAD, 16):
            p2l2[pl.ds(g, 16)] = neg16

        def build_p2l2(g, _):
            acc = neg16
            for j in range(16):
                f = lane * 16 + g * 256 + j
                v = plsc.load_gather(score_buf, [f >> 7, f & 127])
                acc = jnp.maximum(acc, v)
            p2l2[pl.ds(g * 16, 16)] = acc
            return 0

        lax.fori_loop(0, P2_L2N // 16, build_p2l2, 0)

        t0 = _l3_group(p2l2, 0)
        t1 = _l3_group(p2l2, 1)
        t2 = _l3_group(p2l2, 2)
        t3 = _l3_group(p2l2, 3)

        def extract_elem(t, carry):
            t0, t1, t2, t3 = carry
            m = _hmax(jnp.maximum(jnp.maximum(t0, t1), jnp.maximum(t2, t3)))
            in0 = _pc(t0 == m) > 0
            in1 = _pc(t1 == m) > 0
            in2 = _pc(t2 == m) > 0
            g = jnp.where(in0, 0, jnp.where(in1, 1, jnp.where(in2, 2, 3)))
            grp = jnp.where(in0, t0, jnp.where(in1, t1, jnp.where(in2, t2, t3)))
            j = g * 16 + _ffs(grp == m)
            u = p2l2[pl.ds(j * 16, 16)]
            i_off = _ffs(u == m)
            e = j * 16 + i_off
            row = e >> 3
            col = (e & 7) * 16
            w = score_buf[row, pl.ds(col, 16)]
            c_off = _ffs(w == m)
            f = e * 16 + c_off
            _store1(val_buf, t, m, lane)
            _store1(idx_buf, t,
                    _load1(chunk_ids, f >> 7) * CHUNK + (f & 127), lane)
            w2 = jnp.where(lane == c_off, NEG, w)
            score_buf[row, pl.ds(col, 16)] = w2
            u_masked = jnp.where(lane == i_off, NEG, u)
            nvw = _hmax(w2)
            nvu = jnp.maximum(_hmax(u_masked), nvw)
            p2l2[pl.ds(j * 16, 16)] = jnp.where(lane == i_off, nvw, u)
            upd = lane == (j - g * 16)
            t0n = jnp.where(jnp.logical_and(g == 0, upd), nvu, t0)
            t1n = jnp.where(jnp.logical_and(g == 1, upd), nvu, t1)
            t2n = jnp.where(jnp.logical_and(g == 2, upd), nvu, t2)
            t3n = jnp.where(jnp.logical_and(g == 3, upd), nvu, t3)
            return (t0n, t1n, t2n, t3n)

        lax.fori_loop(0, K, extract_elem, (t0, t1, t2, t3))

        pltpu.async_copy(ids_hbm.at[idx_buf], gath_ids, sem).wait()
        pltpu.sync_copy(val_buf, vals_out.at[q])
        pltpu.sync_copy(gath_ids, idx_out.at[q])


def _stage_b(scores3, cmax, identifiers):
    mesh = plsc.VectorSubcoreMesh(core_axis_name="c", subcore_axis_name="s")
    kfn = pl.kernel(
        _sel_body,
        out_type=[
            jax.ShapeDtypeStruct((NQ, 128), jnp.float32),
            jax.ShapeDtypeStruct((NQ, 128), jnp.int32),
        ],
        mesh=mesh,
        scratch_types=[
            pltpu.VMEM((NCHUNK,), jnp.float32),
            pltpu.VMEM((L2N + 16,), jnp.float32),
            pltpu.VMEM((128,), jnp.int32),
            pltpu.VMEM((128, CHUNK), jnp.float32),
            pltpu.VMEM((P2_L2PAD,), jnp.float32),
            pltpu.VMEM((128,), jnp.float32),
            pltpu.VMEM((128,), jnp.int32),
            pltpu.VMEM((128,), jnp.int32),
            pltpu.SemaphoreType.DMA,
        ],
        compiler_params=pltpu.CompilerParams(needs_layout_passes=False),
    )
    return kfn(scores3, cmax, identifiers)


def kernel(queries, candidates, identifiers, k):
    scores3, cmax = _stage_a(queries, candidates.T)
    vals, idx = _stage_b(scores3, cmax, identifiers)
    return (vals[:, :K], idx[:, :K])


# speedup vs baseline: 1.1969x; 1.1002x over previous
import functools

import jax
import jax.numpy as jnp
from jax import lax
from jax.experimental import pallas as pl
from jax.experimental.pallas import tpu as pltpu
from jax.experimental.pallas import tpu_sc as plsc

NQ = 16
ND = 32
NCAND = 1000000
K = 100

BLK = 131072
NBLK = 8
NPAD = NBLK * BLK
CHUNK = 128
NCHUNK = NPAD // CHUNK
L2N = NCHUNK // 16
L3N = 32
P2_L2N = (K * CHUNK) // 16
P2_L2PAD = 1024
P2_L3N = 64

NEG = float("-inf")


def _score_body(q_ref, c_ref, scores_ref, cmax_ref):
    i = pl.program_id(0)
    q = q_ref[...]
    c = c_ref[...]
    s = lax.dot_general(q, c, (((1,), (0,)), ((), ())),
                        preferred_element_type=jnp.float32)
    gidx = i * BLK + lax.broadcasted_iota(jnp.int32, (NQ, BLK), 1)
    s = jnp.where(gidx < NCAND, s, NEG)
    s3 = s.reshape(NQ, BLK // CHUNK, CHUNK)
    scores_ref[...] = s3
    cmax_ref[...] = jnp.max(s3, axis=2)


def _stage_a(queries, candidates):
    return pl.pallas_call(
        _score_body,
        grid=(NBLK,),
        in_specs=[
            pl.BlockSpec((NQ, ND), lambda i: (0, 0)),
            pl.BlockSpec((ND, BLK), lambda i: (0, i)),
        ],
        out_specs=[
            pl.BlockSpec((NQ, BLK // CHUNK, CHUNK), lambda i: (0, i, 0)),
            pl.BlockSpec((NQ, BLK // CHUNK), lambda i: (0, i)),
        ],
        out_shape=[
            jax.ShapeDtypeStruct((NQ, NCHUNK, CHUNK), jnp.float32),
            jax.ShapeDtypeStruct((NQ, NCHUNK), jnp.float32),
        ],
        compiler_params=pltpu.CompilerParams(
            dimension_semantics=("arbitrary",),
            vmem_limit_bytes=61440 * 1024),
    )(queries, candidates)


def _ffsv(mask):
    return plsc.all_reduce_ffs(mask)


def _pcv(mask):
    return plsc.all_reduce_population_count(mask)


def _hmax(v):
    return lax.reduce_max(v, (0,))


def _load1(ref, i):
    return plsc.load_gather(ref, [jnp.full((16,), i, jnp.int32)])[0]


def _store1(ref, i, val, lane):
    idx = jnp.full((16,), i, jnp.int32)
    v = jnp.full((16,), val)
    plsc.store_scatter(ref, [idx], v, mask=lane == 0)


def _sel_body(scores_hbm, cmax_hbm, ids_hbm, vals_out, idx_out,
              cm_buf, l2_buf, chunk_ids, score_buf,
              p2l2, val_buf, idx_buf, gath_ids, sem):
    cid = lax.axis_index("c")
    sid = lax.axis_index("s")
    q = sid
    lane = lax.iota(jnp.int32, 16)
    neg16 = jnp.full((16,), NEG, jnp.float32)

    @pl.when(cid == 0)
    def _():
        pltpu.sync_copy(cmax_hbm.at[q], cm_buf)
        l2_buf[pl.ds(L2N, 16)] = neg16
        for g in range(8):
            chunk_ids[pl.ds(g * 16, 16)] = jnp.zeros((16,), jnp.int32)
            idx_buf[pl.ds(g * 16, 16)] = jnp.zeros((16,), jnp.int32)
            val_buf[pl.ds(g * 16, 16)] = jnp.zeros((16,), jnp.float32)

        def build_l2(g, _):
            acc = neg16
            for j in range(16):
                v = plsc.load_gather(cm_buf, [lane * 16 + g * 256 + j])
                acc = jnp.maximum(acc, v)
            l2_buf[pl.ds(g * 16, 16)] = acc
            return 0

        lax.fori_loop(0, L2N // 16, build_l2, 0)

        def _l3_group(buf, g):
            acc = neg16
            for j in range(16):
                acc = jnp.maximum(
                    acc, plsc.load_gather(buf, [lane * 16 + g * 256 + j]))
            return acc

        v0 = _l3_group(l2_buf, 0)
        v1 = _l3_group(l2_buf, 1)

        def extract_chunk(t, carry):
            v0, v1 = carry
            m = _hmax(jnp.maximum(v0, v1))
            use_hi = _pcv(v0 == m) == 0
            grp = jnp.where(use_hi, v1, v0)
            jv = jnp.where(use_hi, 16, 0) + _ffsv(grp == m)
            u = plsc.load_gather(l2_buf, [jv * 16 + lane])
            iov = _ffsv(u == m)
            iv = jv * 16 + iov
            w = plsc.load_gather(cm_buf, [iv * 16 + lane])
            cov = _ffsv(w == m)
            plsc.store_scatter(chunk_ids, [jnp.full((16,), t, jnp.int32)],
                               iv * 16 + cov, mask=lane == 0)
            w2 = jnp.where(lane == cov, NEG, w)
            plsc.store_scatter(cm_buf, [iv * 16 + lane], w2)
            um = jnp.where(lane == iov, NEG, u)
            nvw = _hmax(w2)
            nvuv = jnp.maximum(jnp.full((16,), _hmax(um), jnp.float32),
                               jnp.full((16,), nvw, jnp.float32))
            plsc.store_scatter(l2_buf, [jv * 16 + lane],
                               jnp.where(lane == iov, nvw, u))
            joff = _ffsv(grp == m)
            upd = lane == joff
            v0n = jnp.where(jnp.logical_and(jnp.logical_not(use_hi), upd),
                            nvuv, v0)
            v1n = jnp.where(jnp.logical_and(use_hi, upd), nvuv, v1)
            return (v0n, v1n)

        lax.fori_loop(0, K, extract_chunk, (v0, v1))

        pltpu.async_copy(scores_hbm.at[q].at[chunk_ids], score_buf, sem).wait()

        for g in range(P2_L2N, P2_L2PAD, 16):
            p2l2[pl.ds(g, 16)] = neg16

        def build_p2l2(g, _):
            acc = neg16
            for j in range(16):
                f = lane * 16 + g * 256 + j
                v = plsc.load_gather(score_buf, [f >> 7, f & 127])
                acc = jnp.maximum(acc, v)
            p2l2[pl.ds(g * 16, 16)] = acc
            return 0

        lax.fori_loop(0, P2_L2N // 16, build_p2l2, 0)

        t0 = _l3_group(p2l2, 0)
        t1 = _l3_group(p2l2, 1)
        t2 = _l3_group(p2l2, 2)
        t3 = _l3_group(p2l2, 3)

        def extract_elem(t, carry):
            t0, t1, t2, t3 = carry
            m = _hmax(jnp.maximum(jnp.maximum(t0, t1), jnp.maximum(t2, t3)))
            p0 = _pcv(t0 == m) > 0
            p1 = _pcv(t1 == m) > 0
            p2 = _pcv(t2 == m) > 0
            gv = jnp.where(p0, 0, jnp.where(p1, 1, jnp.where(p2, 2, 3)))
            grp = jnp.where(p0, t0, jnp.where(p1, t1, jnp.where(p2, t2, t3)))
            joff = _ffsv(grp == m)
            jv = gv * 16 + joff
            u = plsc.load_gather(p2l2, [jv * 16 + lane])
            iov = _ffsv(u == m)
            ev = jv * 16 + iov
            rowv = ev >> 3
            colv = (ev & 7) * 16 + lane
            w = plsc.load_gather(score_buf, [rowv, colv])
            cov = _ffsv(w == m)
            fv = ev * 16 + cov
            tsplat = jnp.full((16,), t, jnp.int32)
            plsc.store_scatter(val_buf, [tsplat],
                               jnp.full((16,), m, jnp.float32), mask=lane == 0)
            cid = plsc.load_gather(chunk_ids, [fv >> 7])
            plsc.store_scatter(idx_buf, [tsplat],
                               cid * CHUNK + (fv & 127), mask=lane == 0)
            w2 = jnp.where(lane == cov, NEG, w)
            plsc.store_scatter(score_buf, [rowv, colv], w2)
            um = jnp.where(lane == iov, NEG, u)
            nvw = _hmax(w2)
            nvuv = jnp.maximum(jnp.full((16,), _hmax(um), jnp.float32),
                               jnp.full((16,), nvw, jnp.float32))
            plsc.store_scatter(p2l2, [jv * 16 + lane],
                               jnp.where(lane == iov, nvw, u))
            upd = lane == joff
            np0 = jnp.logical_not(p0)
            np1 = jnp.logical_not(p1)
            t0n = jnp.where(jnp.logical_and(p0, upd), nvuv, t0)
            t1n = jnp.where(jnp.logical_and(jnp.logical_and(np0, p1), upd),
                            nvuv, t1)
            t2n = jnp.where(
                jnp.logical_and(jnp.logical_and(np0, jnp.logical_and(np1, p2)),
                                upd), nvuv, t2)
            t3n = jnp.where(
                jnp.logical_and(
                    jnp.logical_and(np0, jnp.logical_and(
                        np1, jnp.logical_not(p2))), upd), nvuv, t3)
            return (t0n, t1n, t2n, t3n)

        lax.fori_loop(0, K, extract_elem, (t0, t1, t2, t3))

        pltpu.async_copy(ids_hbm.at[idx_buf], gath_ids, sem).wait()
        pltpu.sync_copy(val_buf, vals_out.at[q])
        pltpu.sync_copy(gath_ids, idx_out.at[q])


def _stage_b(scores3, cmax, identifiers):
    mesh = plsc.VectorSubcoreMesh(core_axis_name="c", subcore_axis_name="s")
    kfn = pl.kernel(
        _sel_body,
        out_type=[
            jax.ShapeDtypeStruct((NQ, 128), jnp.float32),
            jax.ShapeDtypeStruct((NQ, 128), jnp.int32),
        ],
        mesh=mesh,
        scratch_types=[
            pltpu.VMEM((NCHUNK,), jnp.float32),
            pltpu.VMEM((L2N + 16,), jnp.float32),
            pltpu.VMEM((128,), jnp.int32),
            pltpu.VMEM((128, CHUNK), jnp.float32),
            pltpu.VMEM((P2_L2PAD,), jnp.float32),
            pltpu.VMEM((128,), jnp.float32),
            pltpu.VMEM((128,), jnp.int32),
            pltpu.VMEM((128,), jnp.int32),
            pltpu.SemaphoreType.DMA,
        ],
        compiler_params=pltpu.CompilerParams(needs_layout_passes=False),
    )
    return kfn(scores3, cmax, identifiers)


def kernel(queries, candidates, identifiers, k):
    scores3, cmax = _stage_a(queries, candidates.T)
    vals, idx = _stage_b(scores3, cmax, identifiers)
    return (vals[:, :K], idx[:, :K])


# speedup vs baseline: 1.2007x; 1.0032x over previous
import functools

import jax
import jax.numpy as jnp
from jax import lax
from jax.experimental import pallas as pl
from jax.experimental.pallas import tpu as pltpu
from jax.experimental.pallas import tpu_sc as plsc

NQ = 16
ND = 32
NCAND = 1000000
K = 100

BLK = 131072
NBLK = 8
NPAD = NBLK * BLK
CHUNK = 128
NCHUNK = NPAD // CHUNK
L2N = NCHUNK // 16
L3N = 32
P2_L2N = (K * CHUNK) // 16
P2_L2PAD = 1024
P2_L3N = 64

NEG = float("-inf")


def _score_body(q_ref, c_ref, scores_ref, cmax_ref):
    i = pl.program_id(0)
    q = q_ref[...]
    c = c_ref[...]
    s = lax.dot_general(q, c, (((1,), (0,)), ((), ())),
                        preferred_element_type=jnp.float32)
    gidx = i * BLK + lax.broadcasted_iota(jnp.int32, (NQ, BLK), 1)
    s = jnp.where(gidx < NCAND, s, NEG)
    s3 = s.reshape(NQ, BLK // CHUNK, CHUNK)
    scores_ref[...] = s3
    cmax_ref[...] = jnp.max(s3, axis=2)


def _stage_a(queries, candidates):
    return pl.pallas_call(
        _score_body,
        grid=(NBLK,),
        in_specs=[
            pl.BlockSpec((NQ, ND), lambda i: (0, 0)),
            pl.BlockSpec((ND, BLK), lambda i: (0, i)),
        ],
        out_specs=[
            pl.BlockSpec((NQ, BLK // CHUNK, CHUNK), lambda i: (0, i, 0)),
            pl.BlockSpec((NQ, BLK // CHUNK), lambda i: (0, i)),
        ],
        out_shape=[
            jax.ShapeDtypeStruct((NQ, NCHUNK, CHUNK), jnp.float32),
            jax.ShapeDtypeStruct((NQ, NCHUNK), jnp.float32),
        ],
        compiler_params=pltpu.CompilerParams(
            dimension_semantics=("arbitrary",),
            vmem_limit_bytes=61440 * 1024),
    )(queries, candidates)


def _ffsv(mask):
    return plsc.all_reduce_ffs(mask)


def _pcv(mask):
    return plsc.all_reduce_population_count(mask)


def _hmax(v):
    return lax.reduce_max(v, (0,))


def _load1(ref, i):
    return plsc.load_gather(ref, [jnp.full((16,), i, jnp.int32)])[0]


def _store1(ref, i, val, lane):
    idx = jnp.full((16,), i, jnp.int32)
    v = jnp.full((16,), val)
    plsc.store_scatter(ref, [idx], v, mask=lane == 0)


def _sel_body(scores_hbm, cmax_hbm, ids_hbm, vals_out, idx_out,
              cm_buf, l2_buf, chunk_ids, score_buf,
              p2l2, val_buf, idx_buf, gath_ids, sem):
    cid = lax.axis_index("c")
    sid = lax.axis_index("s")
    q = sid
    lane = lax.iota(jnp.int32, 16)
    neg16 = jnp.full((16,), NEG, jnp.float32)

    @pl.when(cid == 0)
    def _():
        pltpu.sync_copy(cmax_hbm.at[q], cm_buf)
        l2_buf[pl.ds(L2N, 16)] = neg16
        for g in range(8):
            chunk_ids[pl.ds(g * 16, 16)] = jnp.zeros((16,), jnp.int32)
            idx_buf[pl.ds(g * 16, 16)] = jnp.zeros((16,), jnp.int32)
            val_buf[pl.ds(g * 16, 16)] = jnp.zeros((16,), jnp.float32)

        def build_l2(g, _):
            acc = neg16
            for j in range(16):
                v = plsc.load_gather(cm_buf, [lane * 16 + g * 256 + j])
                acc = jnp.maximum(acc, v)
            l2_buf[pl.ds(g * 16, 16)] = acc
            return 0

        lax.fori_loop(0, L2N // 16, build_l2, 0)

        def _l3_group(buf, g):
            acc = neg16
            for j in range(16):
                acc = jnp.maximum(
                    acc, plsc.load_gather(buf, [lane * 16 + g * 256 + j]))
            return acc

        v0 = _l3_group(l2_buf, 0)
        v1 = _l3_group(l2_buf, 1)

        def extract_chunk(t, carry):
            v0, v1 = carry
            m = _hmax(jnp.maximum(v0, v1))
            use_hi = _pcv(v0 == m) == 0
            grp = jnp.where(use_hi, v1, v0)
            jv = jnp.where(use_hi, 16, 0) + _ffsv(grp == m)
            u = plsc.load_gather(l2_buf, [jv * 16 + lane])
            iov = _ffsv(u == m)
            iv = jv * 16 + iov
            w = plsc.load_gather(cm_buf, [iv * 16 + lane])
            cov = _ffsv(w == m)
            plsc.store_scatter(chunk_ids, [jnp.full((16,), t, jnp.int32)],
                               iv * 16 + cov, mask=lane == 0)
            w2 = jnp.where(lane == cov, NEG, w)
            plsc.store_scatter(cm_buf, [iv * 16 + lane], w2)
            um = jnp.where(lane == iov, NEG, u)
            nvw = _hmax(w2)
            nvuv = jnp.maximum(jnp.full((16,), _hmax(um), jnp.float32),
                               jnp.full((16,), nvw, jnp.float32))
            plsc.store_scatter(l2_buf, [jv * 16 + lane],
                               jnp.where(lane == iov, nvw, u))
            joff = _ffsv(grp == m)
            upd = lane == joff
            v0n = jnp.where(jnp.logical_and(jnp.logical_not(use_hi), upd),
                            nvuv, v0)
            v1n = jnp.where(jnp.logical_and(use_hi, upd), nvuv, v1)
            return (v0n, v1n)

        carry = lax.fori_loop(0, 64, extract_chunk, (v0, v1))
        cpA = pltpu.async_copy(scores_hbm.at[q].at[chunk_ids.at[pl.ds(0, 64)]],
                               score_buf.at[pl.ds(0, 64)], sem)
        lax.fori_loop(64, K, extract_chunk, carry)

        cpB = pltpu.async_copy(scores_hbm.at[q].at[chunk_ids.at[pl.ds(64, 64)]],
                               score_buf.at[pl.ds(64, 64)], sem)
        cpA.wait()
        cpB.wait()

        for g in range(P2_L2N, P2_L2PAD, 16):
            p2l2[pl.ds(g, 16)] = neg16

        def build_p2l2(g, _):
            acc = neg16
            for j in range(16):
                f = lane * 16 + g * 256 + j
                v = plsc.load_gather(score_buf, [f >> 7, f & 127])
                acc = jnp.maximum(acc, v)
            p2l2[pl.ds(g * 16, 16)] = acc
            return 0

        lax.fori_loop(0, P2_L2N // 16, build_p2l2, 0)

        t0 = _l3_group(p2l2, 0)
        t1 = _l3_group(p2l2, 1)
        t2 = _l3_group(p2l2, 2)
        t3 = _l3_group(p2l2, 3)

        def extract_elem(t, carry):
            t0, t1, t2, t3 = carry
            m = _hmax(jnp.maximum(jnp.maximum(t0, t1), jnp.maximum(t2, t3)))
            p0 = _pcv(t0 == m) > 0
            p1 = _pcv(t1 == m) > 0
            p2 = _pcv(t2 == m) > 0
            gv = jnp.where(p0, 0, jnp.where(p1, 1, jnp.where(p2, 2, 3)))
            grp = jnp.where(p0, t0, jnp.where(p1, t1, jnp.where(p2, t2, t3)))
            joff = _ffsv(grp == m)
            jv = gv * 16 + joff
            u = plsc.load_gather(p2l2, [jv * 16 + lane])
            iov = _ffsv(u == m)
            ev = jv * 16 + iov
            rowv = ev >> 3
            colv = (ev & 7) * 16 + lane
            w = plsc.load_gather(score_buf, [rowv, colv])
            cov = _ffsv(w == m)
            fv = ev * 16 + cov
            tsplat = jnp.full((16,), t, jnp.int32)
            plsc.store_scatter(val_buf, [tsplat],
                               jnp.full((16,), m, jnp.float32), mask=lane == 0)
            cid = plsc.load_gather(chunk_ids, [fv >> 7])
            plsc.store_scatter(idx_buf, [tsplat],
                               cid * CHUNK + (fv & 127), mask=lane == 0)
            w2 = jnp.where(lane == cov, NEG, w)
            plsc.store_scatter(score_buf, [rowv, colv], w2)
            um = jnp.where(lane == iov, NEG, u)
            nvw = _hmax(w2)
            nvuv = jnp.maximum(jnp.full((16,), _hmax(um), jnp.float32),
                               jnp.full((16,), nvw, jnp.float32))
            plsc.store_scatter(p2l2, [jv * 16 + lane],
                               jnp.where(lane == iov, nvw, u))
            upd = lane == joff
            np0 = jnp.logical_not(p0)
            np1 = jnp.logical_not(p1)
            t0n = jnp.where(jnp.logical_and(p0, upd), nvuv, t0)
            t1n = jnp.where(jnp.logical_and(jnp.logical_and(np0, p1), upd),
                            nvuv, t1)
            t2n = jnp.where(
                jnp.logical_and(jnp.logical_and(np0, jnp.logical_and(np1, p2)),
                                upd), nvuv, t2)
            t3n = jnp.where(
                jnp.logical_and(
                    jnp.logical_and(np0, jnp.logical_and(
                        np1, jnp.logical_not(p2))), upd), nvuv, t3)
            return (t0n, t1n, t2n, t3n)

        lax.fori_loop(0, K, extract_elem, (t0, t1, t2, t3))

        pltpu.async_copy(ids_hbm.at[idx_buf], gath_ids, sem).wait()
        pltpu.sync_copy(val_buf, vals_out.at[q])
        pltpu.sync_copy(gath_ids, idx_out.at[q])


def _stage_b(scores3, cmax, identifiers):
    mesh = plsc.VectorSubcoreMesh(core_axis_name="c", subcore_axis_name="s")
    kfn = pl.kernel(
        _sel_body,
        out_type=[
            jax.ShapeDtypeStruct((NQ, 128), jnp.float32),
            jax.ShapeDtypeStruct((NQ, 128), jnp.int32),
        ],
        mesh=mesh,
        scratch_types=[
            pltpu.VMEM((NCHUNK,), jnp.float32),
            pltpu.VMEM((L2N + 16,), jnp.float32),
            pltpu.VMEM((128,), jnp.int32),
            pltpu.VMEM((128, CHUNK), jnp.float32),
            pltpu.VMEM((P2_L2PAD,), jnp.float32),
            pltpu.VMEM((128,), jnp.float32),
            pltpu.VMEM((128,), jnp.int32),
            pltpu.VMEM((128,), jnp.int32),
            pltpu.SemaphoreType.DMA,
        ],
        compiler_params=pltpu.CompilerParams(needs_layout_passes=False),
    )
    return kfn(scores3, cmax, identifiers)


def kernel(queries, candidates, identifiers, k):
    scores3, cmax = _stage_a(queries, candidates.T)
    vals, idx = _stage_b(scores3, cmax, identifiers)
    return (vals[:, :K], idx[:, :K])
